# baseline (speedup 1.0000x reference)
import functools

import jax
import jax.numpy as jnp
from jax import lax
from jax.experimental import pallas as pl
from jax.experimental.pallas import tpu as pltpu


def _s2v_body(xv_ref, ws_ref,
              w1a_ref, b1a_ref, w1b_ref, b1b_ref,
              w2_ref, b2_ref, w3_ref, b3_ref, w4_ref, b4_ref,
              out_ref, *, T, ck):
    _, N, _ = ws_ref.shape
    emb = out_ref.shape[2]
    f32 = jnp.float32
    bf16 = jnp.bfloat16

    def bdot(a, b):
        return jnp.dot(a.astype(bf16), b.astype(bf16),
                       preferred_element_type=f32)

    xv = xv_ref[0]
    h = jnp.maximum(bdot(xv, w1a_ref[...]) + b1a_ref[...], 0.0)
    s1 = bdot(h, w1b_ref[...]) + b1b_ref[...]

    w4v = w4_ref[...].reshape(1, 1, emb)
    b4v = b4_ref[...].reshape(1, 1, emb)

    def s3_step(c, acc):
        i0 = pl.multiple_of(c * ck, ck)
        chunk = ws_ref[0, pl.ds(i0, ck), :]
        t = jnp.maximum(chunk[:, :, None] * w4v + b4v, 0.0)
        return acc + jnp.sum(t, axis=0)

    s3_2 = lax.fori_loop(0, N // ck, s3_step,
                         jnp.zeros((N, emb), f32))
    s3 = bdot(s3_2, w3_ref[...]) + b3_ref[...]

    s13 = s1 + s3 + b2_ref[...]

    ws_b = ws_ref[0].astype(bf16)
    w2_b = w2_ref[...].astype(bf16)

    mu = jnp.maximum(s13, 0.0)
    for _ in range(T - 1):
        mw = jnp.dot(mu.astype(bf16), w2_b, preferred_element_type=f32)
        agg = jnp.dot(ws_b, mw.astype(bf16), preferred_element_type=f32)
        mu = jnp.maximum(s13 + agg, 0.0)

    out_ref[0] = mu


def kernel(xv, Ws, w1a, b1a, w1b, b1b, w2, b2, w3, b3, w4, b4):
    B, N, node_dim = xv.shape
    emb = w1a.shape[1]
    T = 4
    ck = 8

    weight_args = (w1a, b1a, w1b, b1b, w2, b2, w3, b3, w4, b4)

    def bmap(i):
        return (i, 0, 0)

    def wmap(i):
        return (0, 0)

    body = functools.partial(_s2v_body, T=T, ck=ck)
    return pl.pallas_call(
        body,
        out_shape=jax.ShapeDtypeStruct((B, N, emb), jnp.float32),
        grid=(B,),
        in_specs=[
            pl.BlockSpec((1, N, node_dim), bmap),
            pl.BlockSpec((1, N, N), bmap),
        ] + [pl.BlockSpec(w.shape, wmap) for w in weight_args],
        out_specs=pl.BlockSpec((1, N, emb), bmap),
        compiler_params=pltpu.CompilerParams(
            dimension_semantics=("parallel",),
            vmem_limit_bytes=96 * 1024 * 1024),
    )(xv, Ws, *weight_args)


# speedup vs baseline: 1.2077x; 1.2077x over previous
import functools

import jax
import jax.numpy as jnp
from jax import lax
from jax.experimental import pallas as pl
from jax.experimental.pallas import tpu as pltpu


def _s2v_body(xv_ref, ws_ref,
              w1a_ref, b1a_ref, w1b_ref, b1b_ref,
              w2_ref, b2_ref, w3_ref, b3_ref, w4_s, b4_s,
              out_ref, s3t_ref, *, T):
    _, N, _ = ws_ref.shape
    emb = out_ref.shape[2]
    f32 = jnp.float32
    bf16 = jnp.bfloat16

    def bdot(a, b):
        return jnp.dot(a.astype(bf16), b.astype(bf16),
                       preferred_element_type=f32)

    xv = xv_ref[0]
    h = jnp.maximum(bdot(xv, w1a_ref[...]) + b1a_ref[...], 0.0)
    s1 = bdot(h, w1b_ref[...]) + b1b_ref[...]

    ws = ws_ref[0]

    def e_step(e, carry):
        w = w4_s[0, e]
        b = b4_s[0, e]
        t = jnp.maximum(ws * w + b, 0.0)
        s3t_ref[pl.ds(e, 1), :] = jnp.sum(t, axis=0, keepdims=True)
        return carry

    lax.fori_loop(0, emb, e_step, 0)
    s3_2 = s3t_ref[...].T
    s3 = bdot(s3_2, w3_ref[...]) + b3_ref[...]

    s13 = s1 + s3 + b2_ref[...]

    ws_b = ws.astype(bf16)
    w2_b = w2_ref[...].astype(bf16)

    mu = jnp.maximum(s13, 0.0)
    for _ in range(T - 1):
        mw = jnp.dot(mu.astype(bf16), w2_b, preferred_element_type=f32)
        agg = jnp.dot(ws_b, mw.astype(bf16), preferred_element_type=f32)
        mu = jnp.maximum(s13 + agg, 0.0)

    out_ref[0] = mu


def kernel(xv, Ws, w1a, b1a, w1b, b1b, w2, b2, w3, b3, w4, b4):
    B, N, node_dim = xv.shape
    emb = w1a.shape[1]
    T = 4

    def bmap(i):
        return (i, 0, 0)

    def wmap(i):
        return (0, 0)

    vmem_weights = (w1a, b1a, w1b, b1b, w2, b2, w3, b3)

    body = functools.partial(_s2v_body, T=T)
    return pl.pallas_call(
        body,
        out_shape=jax.ShapeDtypeStruct((B, N, emb), jnp.float32),
        grid=(B,),
        in_specs=[
            pl.BlockSpec((1, N, node_dim), bmap),
            pl.BlockSpec((1, N, N), bmap),
        ] + [pl.BlockSpec(w.shape, wmap) for w in vmem_weights] + [
            pl.BlockSpec(memory_space=pltpu.SMEM),
            pl.BlockSpec(memory_space=pltpu.SMEM),
        ],
        out_specs=pl.BlockSpec((1, N, emb), bmap),
        scratch_shapes=[pltpu.VMEM((emb, N), jnp.float32)],
        compiler_params=pltpu.CompilerParams(
            dimension_semantics=("parallel",),
            vmem_limit_bytes=96 * 1024 * 1024),
    )(xv, Ws, *vmem_weights, w4, b4)


# speedup vs baseline: 1.6645x; 1.3782x over previous
import functools

import jax
import jax.numpy as jnp
from jax import lax
from jax.experimental import pallas as pl
from jax.experimental.pallas import tpu as pltpu


def _s2v_body(xv_ref, ws_ref,
              w1a_ref, b1a_ref, w1b_ref, b1b_ref,
              w2_ref, b2_ref, w3_ref, b3_ref, w4_s, b4_s,
              out_ref, s3t_ref, *, T):
    _, N, _ = ws_ref.shape
    emb = out_ref.shape[2]
    f32 = jnp.float32
    bf16 = jnp.bfloat16

    def bdot(a, b):
        return jnp.dot(a.astype(bf16), b.astype(bf16),
                       preferred_element_type=f32)

    xv = xv_ref[0]
    h = jnp.maximum(bdot(xv, w1a_ref[...]) + b1a_ref[...], 0.0)
    s1 = bdot(h, w1b_ref[...]) + b1b_ref[...]

    def e_step(e2, carry):
        e = pl.multiple_of(e2 * 2, 2)
        w0 = w4_s[0, e]
        b0 = b4_s[0, e]
        w1 = w4_s[0, e + 1]
        b1 = b4_s[0, e + 1]
        acc0 = jnp.zeros((8, N), f32)
        acc1 = jnp.zeros((8, N), f32)
        for k in range(N // 8):
            blk = ws_ref[0, k * 8:(k + 1) * 8, :]
            acc0 = acc0 + jnp.maximum(blk * w0 + b0, 0.0)
            acc1 = acc1 + jnp.maximum(blk * w1 + b1, 0.0)
        s3t_ref[pl.ds(e, 1), :] = jnp.sum(acc0, axis=0, keepdims=True)
        s3t_ref[pl.ds(e + 1, 1), :] = jnp.sum(acc1, axis=0, keepdims=True)
        return carry

    lax.fori_loop(0, emb // 2, e_step, 0)
    s3_2 = s3t_ref[...].T
    s3 = bdot(s3_2, w3_ref[...]) + b3_ref[...]

    s13 = s1 + s3 + b2_ref[...]

    ws_b = ws_ref[0].astype(bf16)
    w2_b = w2_ref[...].astype(bf16)

    mu = jnp.maximum(s13, 0.0)
    for _ in range(T - 1):
        mw = jnp.dot(mu.astype(bf16), w2_b, preferred_element_type=f32)
        agg = jnp.dot(ws_b, mw.astype(bf16), preferred_element_type=f32)
        mu = jnp.maximum(s13 + agg, 0.0)

    out_ref[0] = mu


def kernel(xv, Ws, w1a, b1a, w1b, b1b, w2, b2, w3, b3, w4, b4):
    B, N, node_dim = xv.shape
    emb = w1a.shape[1]
    T = 4

    def bmap(i):
        return (i, 0, 0)

    def wmap(i):
        return (0, 0)

    vmem_weights = (w1a, b1a, w1b, b1b, w2, b2, w3, b3)

    body = functools.partial(_s2v_body, T=T)
    return pl.pallas_call(
        body,
        out_shape=jax.ShapeDtypeStruct((B, N, emb), jnp.float32),
        grid=(B,),
        in_specs=[
            pl.BlockSpec((1, N, node_dim), bmap),
            pl.BlockSpec((1, N, N), bmap),
        ] + [pl.BlockSpec(w.shape, wmap) for w in vmem_weights] + [
            pl.BlockSpec(memory_space=pltpu.SMEM),
            pl.BlockSpec(memory_space=pltpu.SMEM),
        ],
        out_specs=pl.BlockSpec((1, N, emb), bmap),
        scratch_shapes=[pltpu.VMEM((emb, N), jnp.float32)],
        compiler_params=pltpu.CompilerParams(
            dimension_semantics=("parallel",),
            vmem_limit_bytes=96 * 1024 * 1024),
    )(xv, Ws, *vmem_weights, w4, b4)
